# baseline (speedup 1.0000x reference)
import functools
import jax
import jax.numpy as jnp
from jax import lax
from jax.experimental import pallas as pl
from jax.experimental.pallas import tpu as pltpu
from jax.experimental.pallas import tpu_sc as plsc

N_NODES_K = 10000
N_EDGES_K = 160000
D_K = 256
H_K = D_K // 2
B_K = 128
NSUB = 16
L = 16
T_BLK = 84
NBLK_PAD = T_BLK * NSUB
E_PAD = NBLK_PAD * B_K
NARR = NBLK_PAD + 3
E_ARR = NARR * B_K
G_CH = 3
N_CH = T_BLK // G_CH
ROWS_PER_SUB = 624

_mesh = plsc.VectorSubcoreMesh(core_axis_name="c", subcore_axis_name="s")


@functools.partial(
    pl.kernel,
    out_type=jax.ShapeDtypeStruct((2, N_NODES_K, H_K), jnp.float32),
    mesh=_mesh,
    scratch_types=[
        pltpu.VMEM((2, G_CH, 1, B_K), jnp.int32),
        pltpu.VMEM((3, B_K), jnp.int32),
        pltpu.VMEM((3, B_K), jnp.float32),
        pltpu.VMEM((3, B_K, H_K), jnp.float32),
        pltpu.VMEM_SHARED((N_NODES_K, H_K), jnp.float32),
        pltpu.SemaphoreType.DMA,
        pltpu.SemaphoreType.DMA,
        pltpu.SemaphoreType.DMA,
        pltpu.SemaphoreType.DMA,
        pltpu.SemaphoreType.DMA,
        pltpu.SemaphoreType.DMA,
        pltpu.SemaphoreType.DMA,
        pltpu.SemaphoreType.DMA,
        pltpu.SemaphoreType.DMA,
        pltpu.SemaphoreType.DMA,
    ],
)
def _spmm_sc(col_h, row_h, val_h, oa_h, ob_h, out_h,
             colb, rowb, valb, mb, acc,
             sem_c, sv0, sv1, sv2, sg0, sg1, sg2, ss0, ss1, ss2):
    c = lax.axis_index("c")
    s = lax.axis_index("s")
    svs = [sv0, sv1, sv2]
    sgs = [sg0, sg1, sg2]
    sss = [ss0, ss1, ss2]
    zeros16 = jnp.zeros((L,), jnp.float32)
    r0 = s * ROWS_PER_SUB
    blk0 = s * T_BLK

    @pl.loop(0, B_K)
    def _(r):
        for jj in range(H_K // L):
            mb[0, r, pl.ds(jj * L, L)] = zeros16

    for kk in range(4):
        pltpu.sync_copy(mb.at[0], acc.at[pl.ds(r0 + kk * B_K, B_K), :])

    @pl.when(s < NSUB - 1)
    def _():
        pltpu.sync_copy(mb.at[0].at[pl.ds(0, 112), :],
                        acc.at[pl.ds(r0 + 4 * B_K, 112), :])

    @pl.when(s == NSUB - 1)
    def _():
        pltpu.sync_copy(mb.at[0], acc.at[pl.ds(r0 + 4 * B_K, B_K), :])

    plsc.subcore_barrier()

    def chunk_load(ch):
        pltpu.async_copy(col_h.at[pl.ds(blk0 + ch * G_CH, G_CH), :, :],
                         colb.at[lax.rem(ch, 2)], sem_c)

    def chunk_wait():
        pltpu.make_async_copy(col_h.at[pl.ds(0, G_CH), :, :],
                              colb.at[0], sem_c).wait()

    def rv_load(t, j):
        pltpu.async_copy(row_h.at[blk0 + t, 0, :], rowb.at[j], svs[j])
        pltpu.async_copy(val_h.at[blk0 + t, 0, :], valb.at[j], svs[j])

    def rv_wait(j):
        pltpu.make_async_copy(row_h.at[0, 0, :], rowb.at[j], svs[j]).wait()
        pltpu.make_async_copy(val_h.at[0, 0, :], valb.at[j], svs[j]).wait()

    def gather_issue(t, j):
        colref = colb.at[lax.rem(t // G_CH, 2), lax.rem(t, G_CH), 0]

        @pl.when(c == 0)
        def _():
            pltpu.async_copy(oa_h.at[colref], mb.at[j], sgs[j])

        @pl.when(c == 1)
        def _():
            pltpu.async_copy(ob_h.at[colref], mb.at[j], sgs[j])

    def gather_wait(j):
        pltpu.make_async_copy(oa_h.at[pl.ds(0, B_K), :],
                              mb.at[j], sgs[j]).wait()

    def scatter_issue(j):
        pltpu.async_copy(mb.at[j], acc.at[rowb.at[j]], sss[j], add=True)

    def scatter_wait(j):
        pltpu.make_async_copy(mb.at[j], acc.at[pl.ds(0, B_K), :],
                              sss[j]).wait()

    chunk_load(0)
    chunk_wait()
    rv_load(0, 0)
    rv_load(1, 1)
    gather_issue(0, 0)
    gather_issue(1, 1)
    pltpu.async_copy(oa_h.at[pl.ds(0, B_K), :], mb.at[2], ss2)

    def block_body(t, b, j, j2):
        if b == 0:
            chunk_load(t // G_CH + 1)
        if b == 1:
            chunk_wait()
        rv_wait(j)
        gather_wait(j)

        @pl.loop(0, B_K // L)
        def _(g):
            vals16 = valb[j, pl.ds(g * L, L)]
            for i in range(L):
                vspl = jnp.take_along_axis(
                    vals16, jnp.full((L,), i, jnp.int32), axis=0)
                e = g * L + i
                for jj in range(H_K // L):
                    sl = pl.ds(jj * L, L)
                    mb[j, e, sl] = mb[j, e, sl] * vspl

        scatter_issue(j)
        scatter_wait(j2)
        gather_issue(t + 2, j2)
        rv_load(t + 2, j2)

    @pl.loop(0, T_BLK // 3)
    def _(m):
        t0 = m * 3
        block_body(t0, 0, 0, 2)
        block_body(t0 + 1, 1, 1, 0)
        block_body(t0 + 2, 2, 2, 1)

    scatter_wait(2)
    gather_wait(0)
    gather_wait(1)
    rv_wait(0)
    rv_wait(1)

    plsc.subcore_barrier()

    for kk in range(4):
        pltpu.sync_copy(acc.at[pl.ds(r0 + kk * B_K, B_K), :],
                        out_h.at[c, pl.ds(r0 + kk * B_K, B_K), :])

    @pl.when(s < NSUB - 1)
    def _():
        pltpu.sync_copy(acc.at[pl.ds(r0 + 4 * B_K, 112), :],
                        out_h.at[c, pl.ds(r0 + 4 * B_K, 112), :])

    @pl.when(s == NSUB - 1)
    def _():
        pltpu.sync_copy(acc.at[pl.ds(r0 + 4 * B_K, B_K), :],
                        out_h.at[c, pl.ds(r0 + 4 * B_K, B_K), :])


def kernel(row, col, value, other):
    padi = jnp.zeros((E_ARR - N_EDGES_K,), jnp.int32)
    padf = jnp.zeros((E_ARR - N_EDGES_K,), jnp.float32)
    row3 = jnp.concatenate([row, padi]).reshape(NARR, 1, B_K)
    col3 = jnp.concatenate([col, padi]).reshape(NARR, 1, B_K)
    val3 = jnp.concatenate([value, padf]).reshape(NARR, 1, B_K)
    oa = other[:, :H_K]
    ob = other[:, H_K:]
    out2 = _spmm_sc(col3, row3, val3, oa, ob)
    return out2.transpose(1, 0, 2).reshape(N_NODES_K, D_K)


# speedup vs baseline: 1.8798x; 1.8798x over previous
import functools
import jax
import jax.numpy as jnp
from jax import lax
from jax.experimental import pallas as pl
from jax.experimental.pallas import tpu as pltpu
from jax.experimental.pallas import tpu_sc as plsc

N_NODES_K = 10000
N_EDGES_K = 160000
D_K = 256
H_K = D_K // 2
B_K = 128
NSUB = 16
L = 16
T_BLK = 80
NBLK_PAD = T_BLK * NSUB
NARR = NBLK_PAD + 1
E_ARR = NARR * B_K
ROWS_PER_SUB = 624

_mesh = plsc.VectorSubcoreMesh(core_axis_name="c", subcore_axis_name="s")


@functools.partial(
    pl.kernel,
    out_type=jax.ShapeDtypeStruct((2, N_NODES_K, H_K), jnp.float32),
    mesh=_mesh,
    scratch_types=[
        pltpu.VMEM((2, B_K), jnp.int32),
        pltpu.VMEM((2, B_K), jnp.int32),
        pltpu.VMEM((2, B_K), jnp.float32),
        pltpu.VMEM((2, B_K), jnp.int32),
        pltpu.VMEM((2, B_K, H_K), jnp.float32),
        pltpu.VMEM_SHARED((N_NODES_K, H_K), jnp.float32),
        pltpu.SemaphoreType.DMA,
        pltpu.SemaphoreType.DMA,
        pltpu.SemaphoreType.DMA,
        pltpu.SemaphoreType.DMA,
    ],
)
def _spmm_sc(col_h, row_h, val_h, oa_h, ob_h, out_h,
             colb, rowb, valb, srow, mb, acc, sem_i, sem_g, ss0, ss1):
    c = lax.axis_index("c")
    s = lax.axis_index("s")
    sss = [ss0, ss1]
    zeros16 = jnp.zeros((L,), jnp.float32)
    r0 = s * ROWS_PER_SUB
    blk0 = s * T_BLK

    @pl.loop(0, B_K)
    def _(r):
        for jj in range(H_K // L):
            mb[0, r, pl.ds(jj * L, L)] = zeros16

    for kk in range(4):
        pltpu.sync_copy(mb.at[0], acc.at[pl.ds(r0 + kk * B_K, B_K), :])

    @pl.when(s < NSUB - 1)
    def _():
        pltpu.sync_copy(mb.at[0].at[pl.ds(0, 112), :],
                        acc.at[pl.ds(r0 + 4 * B_K, 112), :])

    @pl.when(s == NSUB - 1)
    def _():
        pltpu.sync_copy(mb.at[0], acc.at[pl.ds(r0 + 4 * B_K, B_K), :])

    plsc.subcore_barrier()

    def idx_load(t, j):
        pltpu.async_copy(col_h.at[blk0 + t, 0, :], colb.at[j], sem_i)
        pltpu.async_copy(row_h.at[blk0 + t, 0, :], rowb.at[j], sem_i)
        pltpu.async_copy(val_h.at[blk0 + t, 0, :], valb.at[j], sem_i)

    def idx_wait(j):
        pltpu.make_async_copy(col_h.at[0, 0, :], colb.at[j], sem_i).wait()
        pltpu.make_async_copy(row_h.at[0, 0, :], rowb.at[j], sem_i).wait()
        pltpu.make_async_copy(val_h.at[0, 0, :], valb.at[j], sem_i).wait()

    def gather_sync(j):
        @pl.when(c == 0)
        def _():
            pltpu.async_copy(oa_h.at[colb.at[j]], mb.at[j], sem_g)

        @pl.when(c == 1)
        def _():
            pltpu.async_copy(ob_h.at[colb.at[j]], mb.at[j], sem_g)

        pltpu.make_async_copy(oa_h.at[pl.ds(0, B_K), :],
                              mb.at[j], sem_g).wait()

    def scatter_wait(j):
        pltpu.make_async_copy(mb.at[j], acc.at[pl.ds(0, B_K), :],
                              sss[j]).wait()

    idx_load(0, 0)
    pltpu.async_copy(oa_h.at[pl.ds(0, B_K), :], mb.at[0], ss0)
    pltpu.async_copy(oa_h.at[pl.ds(0, B_K), :], mb.at[1], ss1)

    def block_body(t, j):
        scatter_wait(j)
        idx_wait(j)
        idx_load(t + 1, 1 - j)
        gather_sync(j)
        @pl.loop(0, B_K // L)
        def _(g):
            vals16 = valb[j, pl.ds(g * L, L)]
            for i in range(L):
                vspl = jnp.take_along_axis(
                    vals16, jnp.full((L,), i, jnp.int32), axis=0)
                e = g * L + i
                for jj in range(H_K // L):
                    sl = pl.ds(jj * L, L)
                    mb[j, e, sl] = mb[j, e, sl] * vspl

        for g4 in range(B_K // L):
            srow[j, pl.ds(g4 * L, L)] = rowb[j, pl.ds(g4 * L, L)]
        pltpu.async_copy(mb.at[j], acc.at[srow.at[j]], sss[j], add=True)

    @pl.loop(0, T_BLK // 2)
    def _(q):
        block_body(2 * q, 0)
        block_body(2 * q + 1, 1)

    scatter_wait(0)
    scatter_wait(1)
    idx_wait(0)

    plsc.subcore_barrier()

    for kk in range(4):
        pltpu.sync_copy(acc.at[pl.ds(r0 + kk * B_K, B_K), :],
                        out_h.at[c, pl.ds(r0 + kk * B_K, B_K), :])

    @pl.when(s < NSUB - 1)
    def _():
        pltpu.sync_copy(acc.at[pl.ds(r0 + 4 * B_K, 112), :],
                        out_h.at[c, pl.ds(r0 + 4 * B_K, 112), :])

    @pl.when(s == NSUB - 1)
    def _():
        pltpu.sync_copy(acc.at[pl.ds(r0 + 4 * B_K, B_K), :],
                        out_h.at[c, pl.ds(r0 + 4 * B_K, B_K), :])


def kernel(row, col, value, other):
    padi = jnp.zeros((E_ARR - N_EDGES_K,), jnp.int32)
    padf = jnp.zeros((E_ARR - N_EDGES_K,), jnp.float32)
    row3 = jnp.concatenate([row, padi]).reshape(NARR, 1, B_K)
    col3 = jnp.concatenate([col, padi]).reshape(NARR, 1, B_K)
    val3 = jnp.concatenate([value, padf]).reshape(NARR, 1, B_K)
    oa = other[:, :H_K]
    ob = other[:, H_K:]
    out2 = _spmm_sc(col3, row3, val3, oa, ob)
    return out2.transpose(1, 0, 2).reshape(N_NODES_K, D_K)


# speedup vs baseline: 2.4107x; 1.2824x over previous
import jax
import jax.numpy as jnp
from jax import lax
from jax.experimental import pallas as pl
from jax.experimental.pallas import tpu as pltpu
from jax.experimental.pallas import tpu_sc as plsc
import functools

N_NODES_K = 10000
N_EDGES_K = 160000
D_K = 256
H_K = D_K // 2
B_K = 128
NBLK = N_EDGES_K // B_K
NSUB = 16
L = 16
ITERS_PER_SUB = (NBLK + NSUB - 1) // NSUB
ROWS_PER_SUB = 624

_mesh = plsc.VectorSubcoreMesh(core_axis_name="c", subcore_axis_name="s")


@functools.partial(
    pl.kernel,
    out_type=jax.ShapeDtypeStruct((2, N_NODES_K, H_K), jnp.float32),
    mesh=_mesh,
    scratch_types=[
        pltpu.VMEM((B_K,), jnp.int32),
        pltpu.VMEM((B_K,), jnp.int32),
        pltpu.VMEM((B_K,), jnp.float32),
        pltpu.VMEM((B_K, H_K), jnp.float32),
        pltpu.VMEM_SHARED((N_NODES_K, H_K), jnp.float32),
        pltpu.SemaphoreType.DMA,
        pltpu.SemaphoreType.DMA,
    ],
)
def _spmm_sc(row_h, col_h, val_h, oa_h, ob_h, out_h,
             colb, rowb, valb, msg, acc, sem_g, sem_i):
    c = lax.axis_index("c")
    s = lax.axis_index("s")
    zeros16 = jnp.zeros((L,), jnp.float32)

    @pl.loop(0, B_K)
    def _(r):
        for j in range(H_K // L):
            msg[r, pl.ds(j * L, L)] = zeros16

    r0 = s * ROWS_PER_SUB
    for kk in range(4):
        pltpu.sync_copy(msg, acc.at[pl.ds(r0 + kk * B_K, B_K), :])

    @pl.when(s < NSUB - 1)
    def _():
        pltpu.sync_copy(msg.at[pl.ds(0, 112), :],
                        acc.at[pl.ds(r0 + 4 * B_K, 112), :])

    @pl.when(s == NSUB - 1)
    def _():
        pltpu.sync_copy(msg, acc.at[pl.ds(r0 + 4 * B_K, B_K), :])

    plsc.subcore_barrier()

    @pl.loop(0, ITERS_PER_SUB)
    def _(k):
        b = s + k * NSUB

        @pl.when(b < NBLK)
        def _():
            base = b * B_K
            d1 = pltpu.async_copy(col_h.at[pl.ds(base, B_K)], colb, sem_i)
            d2 = pltpu.async_copy(row_h.at[pl.ds(base, B_K)], rowb, sem_i)
            d3 = pltpu.async_copy(val_h.at[pl.ds(base, B_K)], valb, sem_i)
            d1.wait()
            d2.wait()
            d3.wait()

            @pl.when(c == 0)
            def _():
                pltpu.async_copy(oa_h.at[colb], msg, sem_g).wait()

            @pl.when(c == 1)
            def _():
                pltpu.async_copy(ob_h.at[colb], msg, sem_g).wait()

            @pl.loop(0, B_K // L)
            def _(g):
                vals16 = valb[pl.ds(g * L, L)]
                for i in range(L):
                    vspl = jnp.take_along_axis(
                        vals16, jnp.full((L,), i, jnp.int32), axis=0)
                    e = g * L + i
                    for j in range(H_K // L):
                        sl = pl.ds(j * L, L)
                        msg[e, sl] = msg[e, sl] * vspl

            pltpu.sync_copy(msg, acc.at[rowb], add=True)

    plsc.subcore_barrier()

    for kk in range(4):
        pltpu.sync_copy(acc.at[pl.ds(r0 + kk * B_K, B_K), :],
                        out_h.at[c, pl.ds(r0 + kk * B_K, B_K), :])

    @pl.when(s < NSUB - 1)
    def _():
        pltpu.sync_copy(acc.at[pl.ds(r0 + 4 * B_K, 112), :],
                        out_h.at[c, pl.ds(r0 + 4 * B_K, 112), :])

    @pl.when(s == NSUB - 1)
    def _():
        pltpu.sync_copy(acc.at[pl.ds(r0 + 4 * B_K, B_K), :],
                        out_h.at[c, pl.ds(r0 + 4 * B_K, B_K), :])


def kernel(row, col, value, other):
    oa = other[:, :H_K]
    ob = other[:, H_K:]
    out2 = _spmm_sc(row, col, value, oa, ob)
    return out2.transpose(1, 0, 2).reshape(N_NODES_K, D_K)
